# baseline (speedup 1.0000x reference)
import functools
import math

import jax
import jax.numpy as jnp
from jax import lax
from jax.experimental import pallas as pl
from jax.experimental.pallas import tpu as pltpu
from jax.experimental.pallas import tpu_sc as plsc

_EMB_SCALE = math.sqrt(128.0)

_NC = 2
_NS = 16
_NW = _NC * _NS
_CHUNK = 256
_LANES = 16


def _sc_gather_scale(table, tokens_flat):
    b = tokens_flat.shape[0]
    v, d = table.shape
    assert b % (_NW * _CHUNK) == 0
    b_per_w = b // _NW
    n_chunks = b_per_w // _CHUNK

    mesh = plsc.VectorSubcoreMesh(core_axis_name="c", subcore_axis_name="s")

    def _scale_chunk(buf):
        @pl.loop(0, _CHUNK)
        def _(r):
            for c0 in range(0, d, _LANES):
                buf[r, pl.ds(c0, _LANES)] = (
                    buf[r, pl.ds(c0, _LANES)] * _EMB_SCALE)

    nbuf = 3
    rem = n_chunks % nbuf
    main = n_chunks - rem

    @functools.partial(
        pl.kernel,
        mesh=mesh,
        out_type=jax.ShapeDtypeStruct((b, d), jnp.float32),
        scratch_types=(
            [pltpu.VMEM((b_per_w,), jnp.int32)]
            + [pltpu.VMEM((_CHUNK, d), jnp.float32)] * nbuf
            + [pltpu.SemaphoreType.DMA] * (2 * nbuf)
        ),
    )
    def k(tab_hbm, tok_hbm, out_hbm, idx_v, *scratch):
        bufs = scratch[:nbuf]
        gsem = scratch[nbuf:2 * nbuf]
        osem = scratch[2 * nbuf:]
        wid = lax.axis_index("s") * _NC + lax.axis_index("c")
        base = wid * b_per_w
        pltpu.sync_copy(tok_hbm.at[pl.ds(base, b_per_w)], idx_v)

        def gather(chunk, j):
            pltpu.async_copy(
                tab_hbm.at[idx_v.at[pl.ds(chunk * _CHUNK, _CHUNK)]],
                bufs[j], gsem[j])

        def wait_gather(j):
            pltpu.make_async_copy(
                tab_hbm.at[idx_v.at[pl.ds(0, _CHUNK)]], bufs[j],
                gsem[j]).wait()

        def writeout(chunk, j):
            pltpu.async_copy(
                bufs[j], out_hbm.at[pl.ds(base + chunk * _CHUNK, _CHUNK)],
                osem[j])

        def wait_writeout(j):
            pltpu.make_async_copy(
                bufs[j], out_hbm.at[pl.ds(base, _CHUNK)], osem[j]).wait()

        for j in range(nbuf):
            gather(j, j)

        @pl.loop(0, main, step=nbuf)
        def _(c):
            for j in range(nbuf):
                wait_gather(j)
                _scale_chunk(bufs[j])
                writeout(c + j, j)
                @pl.when(c + j + nbuf < n_chunks)
                def _():
                    wait_writeout(j)
                    gather(c + j + nbuf, j)

        for j in range(rem):
            wait_gather(j)
            _scale_chunk(bufs[j])
            writeout(main + j, j)

        for j in range(nbuf):
            wait_writeout(j)

    return k(table, tokens_flat)


@jax.jit
def kernel(tokens, table):
    n, s = tokens.shape
    tokens_perm = tokens.T.reshape(n * s).astype(jnp.int32)
    out = _sc_gather_scale(table, tokens_perm)
    return out.reshape(s, n, table.shape[1]).transpose(1, 0, 2)


# speedup vs baseline: 1.0106x; 1.0106x over previous
import functools
import math

import jax
import jax.numpy as jnp
from jax import lax
from jax.experimental import pallas as pl
from jax.experimental.pallas import tpu as pltpu
from jax.experimental.pallas import tpu_sc as plsc

_EMB_SCALE = math.sqrt(128.0)

_NC = 2
_NS = 16
_NW = _NC * _NS
_CHUNK = 80
_LANES = 16


def _sc_gather_scale(table, tokens_flat):
    b = tokens_flat.shape[0]
    v, d = table.shape
    assert b % (_NW * _CHUNK) == 0
    b_per_w = b // _NW
    n_chunks = b_per_w // _CHUNK

    mesh = plsc.VectorSubcoreMesh(core_axis_name="c", subcore_axis_name="s")

    def _scale_chunk(buf):
        @pl.loop(0, _CHUNK)
        def _(r):
            for c0 in range(0, d, _LANES):
                buf[r, pl.ds(c0, _LANES)] = (
                    buf[r, pl.ds(c0, _LANES)] * _EMB_SCALE)

    nbuf = 8
    rem = n_chunks % nbuf
    main = n_chunks - rem

    @functools.partial(
        pl.kernel,
        mesh=mesh,
        out_type=jax.ShapeDtypeStruct((b, d), jnp.float32),
        scratch_types=(
            [pltpu.VMEM((b_per_w,), jnp.int32)]
            + [pltpu.VMEM((_CHUNK, d), jnp.float32)] * nbuf
            + [pltpu.SemaphoreType.DMA] * (2 * nbuf)
        ),
    )
    def k(tab_hbm, tok_hbm, out_hbm, idx_v, *scratch):
        bufs = scratch[:nbuf]
        gsem = scratch[nbuf:2 * nbuf]
        osem = scratch[2 * nbuf:]
        wid = lax.axis_index("s") * _NC + lax.axis_index("c")
        base = wid * b_per_w
        head = nbuf * _CHUNK
        pltpu.sync_copy(tok_hbm.at[pl.ds(base, head)],
                        idx_v.at[pl.ds(0, head)])

        def gather(chunk, j):
            pltpu.async_copy(
                tab_hbm.at[idx_v.at[pl.ds(chunk * _CHUNK, _CHUNK)]],
                bufs[j], gsem[j])

        def wait_gather(j):
            pltpu.make_async_copy(
                tab_hbm.at[idx_v.at[pl.ds(0, _CHUNK)]], bufs[j],
                gsem[j]).wait()

        def writeout(chunk, j):
            pltpu.async_copy(
                bufs[j], out_hbm.at[pl.ds(base + chunk * _CHUNK, _CHUNK)],
                osem[j])

        def wait_writeout(j):
            pltpu.make_async_copy(
                bufs[j], out_hbm.at[pl.ds(base, _CHUNK)], osem[j]).wait()

        for j in range(nbuf):
            gather(j, j)
        pltpu.sync_copy(tok_hbm.at[pl.ds(base + head, b_per_w - head)],
                        idx_v.at[pl.ds(head, b_per_w - head)])

        @pl.loop(0, main, step=nbuf)
        def _(c):
            for j in range(nbuf):
                wait_gather(j)
                _scale_chunk(bufs[j])
                writeout(c + j, j)
                @pl.when(c + j + nbuf < n_chunks)
                def _():
                    wait_writeout(j)
                    gather(c + j + nbuf, j)

        for j in range(rem):
            wait_gather(j)
            _scale_chunk(bufs[j])
            writeout(main + j, j)

        for j in range(nbuf):
            wait_writeout(j)

    return k(table, tokens_flat)


@jax.jit
def kernel(tokens, table):
    n, s = tokens.shape
    tokens_perm = tokens.T.reshape(n * s).astype(jnp.int32)
    out = _sc_gather_scale(table, tokens_perm)
    return out.reshape(s, n, table.shape[1]).transpose(1, 0, 2)


# speedup vs baseline: 1.0272x; 1.0165x over previous
import functools
import math

import jax
import jax.numpy as jnp
from jax import lax
from jax.experimental import pallas as pl
from jax.experimental.pallas import tpu as pltpu
from jax.experimental.pallas import tpu_sc as plsc

_EMB_SCALE = math.sqrt(128.0)

_NC = 2
_NS = 16
_NW = _NC * _NS
_CHUNK = 80
_LANES = 16


def _sc_gather_scale(table, tokens_flat):
    b = tokens_flat.shape[0]
    v, d = table.shape
    assert b % (_NW * _CHUNK) == 0
    b_per_w = b // _NW
    n_chunks = b_per_w // _CHUNK

    mesh = plsc.VectorSubcoreMesh(core_axis_name="c", subcore_axis_name="s")

    def _scale_chunk(buf):
        @pl.loop(0, _CHUNK)
        def _(r):
            for c0 in range(0, d, _LANES):
                buf[r, pl.ds(c0, _LANES)] = (
                    buf[r, pl.ds(c0, _LANES)] * _EMB_SCALE)

    nbuf = 8
    rem = n_chunks % nbuf
    main = n_chunks - rem

    @functools.partial(
        pl.kernel,
        mesh=mesh,
        out_type=jax.ShapeDtypeStruct((b, d), jnp.float32),
        scratch_types=(
            [pltpu.VMEM((b_per_w,), jnp.int32)]
            + [pltpu.VMEM((_CHUNK, d), jnp.float32)] * nbuf
            + [pltpu.SemaphoreType.DMA] * (2 * nbuf)
        ),
    )
    def k(tab_hbm, tok_hbm, out_hbm, idx_v, *scratch):
        bufs = scratch[:nbuf]
        gsem = scratch[nbuf:2 * nbuf]
        osem = scratch[2 * nbuf:]
        wid = lax.axis_index("s") * _NC + lax.axis_index("c")
        base = wid * b_per_w
        pltpu.sync_copy(tok_hbm.at[pl.ds(base, b_per_w)], idx_v)

        def gather(chunk, j):
            pltpu.async_copy(
                tab_hbm.at[idx_v.at[pl.ds(chunk * _CHUNK, _CHUNK)]],
                bufs[j], gsem[j])

        def wait_gather(j):
            pltpu.make_async_copy(
                tab_hbm.at[idx_v.at[pl.ds(0, _CHUNK)]], bufs[j],
                gsem[j]).wait()

        def writeout(chunk, j):
            pltpu.async_copy(
                bufs[j], out_hbm.at[pl.ds(base + chunk * _CHUNK, _CHUNK)],
                osem[j])

        def wait_writeout(j):
            pltpu.make_async_copy(
                bufs[j], out_hbm.at[pl.ds(base, _CHUNK)], osem[j]).wait()

        for j in range(nbuf):
            gather(j, j)

        @pl.loop(0, main, step=nbuf)
        def _(c):
            for j in range(nbuf):
                wait_gather(j)
                _scale_chunk(bufs[j])
                writeout(c + j, j)
                @pl.when(c + j + nbuf < n_chunks)
                def _():
                    wait_writeout(j)
                    gather(c + j + nbuf, j)

        for j in range(rem):
            wait_gather(j)
            _scale_chunk(bufs[j])
            writeout(main + j, j)

        for j in range(nbuf):
            wait_writeout(j)

    return k(table, tokens_flat)


@jax.jit
def kernel(tokens, table):
    n, s = tokens.shape
    tokens_perm = tokens.T.reshape(n * s).astype(jnp.int32)
    out = _sc_gather_scale(table, tokens_perm)
    return out.reshape(s, n, table.shape[1]).transpose(1, 0, 2)
